# baseline (speedup 1.0000x reference)
import jax
import jax.numpy as jnp
from jax.experimental import pallas as pl
from jax.experimental.pallas import tpu as pltpu

_T, _D, _H, _E = 2048, 1024, 2048, 8
_TB = 128


def _gate_body(x_ref, wg_ref, cwt_ref):
    logits = jax.lax.dot_general(
        wg_ref[...], x_ref[...], (((1,), (1,)), ((), ())),
        preferred_element_type=jnp.float32)
    ei = jax.lax.broadcasted_iota(jnp.int32, logits.shape, 0)
    m1 = jnp.max(logits, axis=0, keepdims=True)
    a1 = jnp.min(jnp.where(logits == m1, ei, _E), axis=0, keepdims=True)
    l2 = jnp.where(ei == a1, -jnp.inf, logits)
    m2 = jnp.max(l2, axis=0, keepdims=True)
    a2 = jnp.min(jnp.where(l2 == m2, ei, _E), axis=0, keepdims=True)
    w1 = 1.0 / (1.0 + jnp.exp(m2 - m1))
    w2 = 1.0 - w1
    cwt_ref[...] = jnp.where(ei == a1, w1, jnp.where(ei == a2, w2, 0.0))


def _ffn_body(cwt_ref, xb_ref, w1_ref, w3_ref, w2_ref, out_ref):
    e = pl.program_id(0)
    t = pl.program_id(1)
    xb = xb_ref[...]
    a = jax.lax.dot_general(xb, w1_ref[0], (((1,), (1,)), ((), ())),
                            preferred_element_type=jnp.float32)
    b = jax.lax.dot_general(xb, w3_ref[0], (((1,), (1,)), ((), ())),
                            preferred_element_type=jnp.float32)
    h = (a / (1.0 + jnp.exp(-a))) * b
    y = jax.lax.dot_general(h, w2_ref[0], (((1,), (1,)), ((), ())),
                            preferred_element_type=jnp.float32)
    y = y * cwt_ref[0, 0][:, None]
    sl = pl.ds(t * _TB, _TB)

    @pl.when(e == 0)
    def _():
        out_ref[sl, :] = y

    @pl.when(e > 0)
    def _():
        out_ref[sl, :] = out_ref[sl, :] + y


def kernel(x, Wg, W1, W2, W3):
    cwt = pl.pallas_call(
        _gate_body,
        out_shape=jax.ShapeDtypeStruct((_E, _T), jnp.float32),
    )(x, Wg)
    cwt3 = cwt.reshape(_E * (_T // _TB), 1, _TB)

    out = pl.pallas_call(
        _ffn_body,
        grid=(_E, _T // _TB),
        in_specs=[
            pl.BlockSpec((1, 1, _TB), lambda e, t: (e * (_T // _TB) + t, 0, 0)),
            pl.BlockSpec((_TB, _D), lambda e, t: (t, 0)),
            pl.BlockSpec((1, _H, _D), lambda e, t: (e, 0, 0)),
            pl.BlockSpec((1, _H, _D), lambda e, t: (e, 0, 0)),
            pl.BlockSpec((1, _D, _H), lambda e, t: (e, 0, 0)),
        ],
        out_specs=pl.BlockSpec((_T, _D), lambda e, t: (0, 0)),
        out_shape=jax.ShapeDtypeStruct((_T, _D), jnp.float32),
        compiler_params=pltpu.CompilerParams(
            dimension_semantics=("arbitrary", "arbitrary")),
    )(cwt3, x, W1, W3, W2)
    return out


# speedup vs baseline: 1.0044x; 1.0044x over previous
import jax
import jax.numpy as jnp
from jax.experimental import pallas as pl
from jax.experimental.pallas import tpu as pltpu

_T, _D, _H, _E = 2048, 1024, 2048, 8
_TB = 128


def _gate_body(x_ref, wg_ref, cwt_ref):
    logits = jax.lax.dot_general(
        wg_ref[...], x_ref[...], (((1,), (1,)), ((), ())),
        preferred_element_type=jnp.float32, precision=jax.lax.Precision.DEFAULT)
    ei = jax.lax.broadcasted_iota(jnp.int32, logits.shape, 0)
    m1 = jnp.max(logits, axis=0, keepdims=True)
    a1 = jnp.min(jnp.where(logits == m1, ei, _E), axis=0, keepdims=True)
    l2 = jnp.where(ei == a1, -jnp.inf, logits)
    m2 = jnp.max(l2, axis=0, keepdims=True)
    a2 = jnp.min(jnp.where(l2 == m2, ei, _E), axis=0, keepdims=True)
    w1 = 1.0 / (1.0 + jnp.exp(m2 - m1))
    w2 = 1.0 - w1
    cwt_ref[...] = jnp.where(ei == a1, w1, jnp.where(ei == a2, w2, 0.0))


def _ffn_body(cwt_ref, xb_ref, w1_ref, w3_ref, w2_ref, out_ref):
    e = pl.program_id(0)
    t = pl.program_id(1)
    xb = xb_ref[...]
    a = jax.lax.dot_general(xb, w1_ref[0], (((1,), (1,)), ((), ())),
                            preferred_element_type=jnp.float32, precision=jax.lax.Precision.DEFAULT)
    b = jax.lax.dot_general(xb, w3_ref[0], (((1,), (1,)), ((), ())),
                            preferred_element_type=jnp.float32, precision=jax.lax.Precision.DEFAULT)
    h = (a / (1.0 + jnp.exp(-a))) * b
    y = jax.lax.dot_general(h, w2_ref[0], (((1,), (1,)), ((), ())),
                            preferred_element_type=jnp.float32, precision=jax.lax.Precision.DEFAULT)
    y = y * cwt_ref[0, 0][:, None]
    sl = pl.ds(t * _TB, _TB)

    @pl.when(e == 0)
    def _():
        out_ref[sl, :] = y

    @pl.when(e > 0)
    def _():
        out_ref[sl, :] = out_ref[sl, :] + y


def kernel(x, Wg, W1, W2, W3):
    cwt = pl.pallas_call(
        _gate_body,
        out_shape=jax.ShapeDtypeStruct((_E, _T), jnp.float32),
    )(x, Wg)
    cwt3 = cwt.reshape(_E * (_T // _TB), 1, _TB)

    out = pl.pallas_call(
        _ffn_body,
        grid=(_E, _T // _TB),
        in_specs=[
            pl.BlockSpec((1, 1, _TB), lambda e, t: (e * (_T // _TB) + t, 0, 0)),
            pl.BlockSpec((_TB, _D), lambda e, t: (t, 0)),
            pl.BlockSpec((1, _H, _D), lambda e, t: (e, 0, 0)),
            pl.BlockSpec((1, _H, _D), lambda e, t: (e, 0, 0)),
            pl.BlockSpec((1, _D, _H), lambda e, t: (e, 0, 0)),
        ],
        out_specs=pl.BlockSpec((_T, _D), lambda e, t: (0, 0)),
        out_shape=jax.ShapeDtypeStruct((_T, _D), jnp.float32),
        compiler_params=pltpu.CompilerParams(
            dimension_semantics=("arbitrary", "arbitrary")),
    )(cwt3, x, W1, W3, W2)
    return out


# speedup vs baseline: 1.4497x; 1.4434x over previous
import jax
import jax.numpy as jnp
from jax.experimental import pallas as pl
from jax.experimental.pallas import tpu as pltpu

_T, _D, _H, _E = 2048, 1024, 2048, 8
_TB = 256
_HB = 512


def _gate_body(x_ref, wg_ref, cwt_ref, x16_ref):
    logits = jax.lax.dot_general(
        wg_ref[...], x_ref[...], (((1,), (1,)), ((), ())),
        preferred_element_type=jnp.float32)
    ei = jax.lax.broadcasted_iota(jnp.int32, logits.shape, 0)
    m1 = jnp.max(logits, axis=0, keepdims=True)
    a1 = jnp.min(jnp.where(logits == m1, ei, _E), axis=0, keepdims=True)
    l2 = jnp.where(ei == a1, -jnp.inf, logits)
    m2 = jnp.max(l2, axis=0, keepdims=True)
    a2 = jnp.min(jnp.where(l2 == m2, ei, _E), axis=0, keepdims=True)
    w1 = 1.0 / (1.0 + jnp.exp(m2 - m1))
    w2 = 1.0 - w1
    cwt_ref[...] = jnp.where(ei == a1, w1, jnp.where(ei == a2, w2, 0.0))
    x16_ref[...] = x_ref[...].astype(jnp.bfloat16)


def _ffn_body(cwt_ref, xb_ref, w1_ref, w3_ref, w2_ref, out_ref,
              w1b, w3b, w2b):
    e = pl.program_id(0)
    h = pl.program_id(1)
    t = pl.program_id(2)

    @pl.when(t == 0)
    def _():
        w1b[...] = w1_ref[0].astype(jnp.bfloat16)
        w3b[...] = w3_ref[0].astype(jnp.bfloat16)
        w2b[...] = w2_ref[0].astype(jnp.bfloat16)

    xb = xb_ref[...]
    a = jax.lax.dot_general(xb, w1b[...], (((1,), (1,)), ((), ())),
                            preferred_element_type=jnp.float32)
    b = jax.lax.dot_general(xb, w3b[...], (((1,), (1,)), ((), ())),
                            preferred_element_type=jnp.float32)
    hh = ((a / (1.0 + jnp.exp(-a))) * b).astype(jnp.bfloat16)
    y = jax.lax.dot_general(hh, w2b[...], (((1,), (1,)), ((), ())),
                            preferred_element_type=jnp.float32)
    y = y * cwt_ref[0, 0][:, None]
    sl = pl.ds(t * _TB, _TB)

    @pl.when(jnp.logical_and(e == 0, h == 0))
    def _():
        out_ref[sl, :] = y

    @pl.when(jnp.logical_or(e > 0, h > 0))
    def _():
        out_ref[sl, :] = out_ref[sl, :] + y


def kernel(x, Wg, W1, W2, W3):
    cwt, x16 = pl.pallas_call(
        _gate_body,
        out_shape=(jax.ShapeDtypeStruct((_E, _T), jnp.float32),
                   jax.ShapeDtypeStruct((_T, _D), jnp.bfloat16)),
    )(x, Wg)
    cwt3 = cwt.reshape(_E * (_T // _TB), 1, _TB)

    nh = _H // _HB
    nt = _T // _TB
    out = pl.pallas_call(
        _ffn_body,
        grid=(_E, nh, nt),
        in_specs=[
            pl.BlockSpec((1, 1, _TB), lambda e, h, t: (e * nt + t, 0, 0)),
            pl.BlockSpec((_TB, _D), lambda e, h, t: (t, 0)),
            pl.BlockSpec((1, _HB, _D), lambda e, h, t: (e, h, 0)),
            pl.BlockSpec((1, _HB, _D), lambda e, h, t: (e, h, 0)),
            pl.BlockSpec((1, _D, _HB), lambda e, h, t: (e, 0, h)),
        ],
        out_specs=pl.BlockSpec((_T, _D), lambda e, h, t: (0, 0)),
        out_shape=jax.ShapeDtypeStruct((_T, _D), jnp.float32),
        scratch_shapes=[
            pltpu.VMEM((_HB, _D), jnp.bfloat16),
            pltpu.VMEM((_HB, _D), jnp.bfloat16),
            pltpu.VMEM((_D, _HB), jnp.bfloat16),
        ],
        compiler_params=pltpu.CompilerParams(
            dimension_semantics=("arbitrary", "arbitrary", "arbitrary")),
    )(cwt3, x16, W1, W3, W2)
    return out
